# Initial kernel scaffold; baseline (speedup 1.0000x reference)
import functools

import jax
import jax.numpy as jnp
from jax import lax
from jax.experimental import pallas as pl
from jax.experimental.pallas import tpu as pltpu
from jax.experimental.pallas import tpu_sc as plsc

N = 10000
E = 320000
D = 128

NC = 2
NS = 16
L = 16
CH = 80

R = 1000
NP = 10240
PT = NP // NS

_HIGH = lax.Precision.HIGHEST


def _dot(a, b):
    return jnp.dot(a, b, precision=_HIGH, preferred_element_type=jnp.float32)


def _tc1_body(xi, xj, w1, asr, adr, ht, hb, aux):
    h = _dot(xi[...], w1[:D, :]) + _dot(xj[...], w1[D:, :])
    ht[...] = h[:, :D]
    hb[...] = h[:, D:]
    a = jnp.sum(h * asr[...], axis=1)
    b = jnp.sum(h * adr[...], axis=1)
    aux[...] = jnp.concatenate(
        [a[None, :], b[None, :], jnp.zeros((6, R), jnp.float32)], axis=0)


def _tc1(xi, xj, w1, asr, adr):
    return pl.pallas_call(
        _tc1_body,
        grid=(N // R,),
        in_specs=[
            pl.BlockSpec((R, D), lambda i: (i, 0)),
            pl.BlockSpec((R, D), lambda i: (i, 0)),
            pl.BlockSpec((2 * D, 2 * D), lambda i: (0, 0)),
            pl.BlockSpec((1, 2 * D), lambda i: (0, 0)),
            pl.BlockSpec((1, 2 * D), lambda i: (0, 0)),
        ],
        out_specs=[
            pl.BlockSpec((R, D), lambda i: (i, 0)),
            pl.BlockSpec((R, D), lambda i: (i, 0)),
            pl.BlockSpec((8, R), lambda i: (0, i)),
        ],
        out_shape=[
            jax.ShapeDtypeStruct((N, D), jnp.float32),
            jax.ShapeDtypeStruct((N, D), jnp.float32),
            jax.ShapeDtypeStruct((8, N), jnp.float32),
        ],
    )(xi, xj, w1, asr, adr)


def _tc2_body(accA, accB, den, b1, w2, asr, adr, h2, aux):
    d = den[0, :][:, None] + 1e-16
    x2a = jax.nn.relu(accA[...] / d + b1[0, :D])
    x2b = jax.nn.relu(accB[...] / d + b1[0, D:])
    h = _dot(x2a, w2[:D, :]) + _dot(x2b, w2[D:, :])
    h2[...] = h
    a = jnp.sum(h * asr[...], axis=1)
    b = jnp.sum(h * adr[...], axis=1)
    aux[...] = jnp.concatenate(
        [a[None, :], b[None, :], jnp.zeros((6, R), jnp.float32)], axis=0)


def _tc2(acc1, den1, b1, w2, asr, adr):
    return pl.pallas_call(
        _tc2_body,
        grid=(N // R,),
        in_specs=[
            pl.BlockSpec((R, D), lambda i: (i, 0)),
            pl.BlockSpec((R, D), lambda i: (N // R + i, 0)),
            pl.BlockSpec((1, R), lambda i: (0, i)),
            pl.BlockSpec((1, 2 * D), lambda i: (0, 0)),
            pl.BlockSpec((2 * D, D), lambda i: (0, 0)),
            pl.BlockSpec((1, D), lambda i: (0, 0)),
            pl.BlockSpec((1, D), lambda i: (0, 0)),
        ],
        out_specs=[
            pl.BlockSpec((R, D), lambda i: (i, 0)),
            pl.BlockSpec((8, R), lambda i: (0, i)),
        ],
        out_shape=[
            jax.ShapeDtypeStruct((N, D), jnp.float32),
            jax.ShapeDtypeStruct((8, N), jnp.float32),
        ],
    )(acc1, acc1, den1, b1, w2, asr, adr)


def _tc3_body(accA, accB, den, b2, w3, as3, ad3, aux):
    d = (den[0, :] + den[1, :])[:, None] + 1e-16
    x3 = jax.nn.relu((accA[...] + accB[...]) / d + b2[0, :])
    h3 = jnp.sum(x3 * w3[...], axis=1)
    aux[...] = jnp.concatenate(
        [h3[None, :], (h3 * as3[0, 0])[None, :], (h3 * ad3[0, 0])[None, :],
         jnp.zeros((5, R), jnp.float32)], axis=0)


def _tc3(acc2, den2, b2, w3, as3, ad3):
    return pl.pallas_call(
        _tc3_body,
        grid=(N // R,),
        in_specs=[
            pl.BlockSpec((R, D), lambda i: (i, 0)),
            pl.BlockSpec((R, D), lambda i: (N // R + i, 0)),
            pl.BlockSpec((2, R), lambda i: (0, i)),
            pl.BlockSpec((1, D), lambda i: (0, 0)),
            pl.BlockSpec((1, D), lambda i: (0, 0)),
            pl.BlockSpec((1, 1), lambda i: (0, 0)),
            pl.BlockSpec((1, 1), lambda i: (0, 0)),
        ],
        out_specs=[pl.BlockSpec((8, R), lambda i: (0, i))],
        out_shape=[jax.ShapeDtypeStruct((8, N), jnp.float32)],
    )(acc2, acc2, den2, b2, w3, as3, ad3)


def _tc4_body(den, acc, b3, out):
    d = den[0, :] + den[1, :] + 1e-16
    a = acc[0, :] + acc[1, :]
    out[...] = jax.nn.sigmoid(a / d + b3[0, 0])[None, :]


def _tc4(den3, acc3, b3):
    return pl.pallas_call(
        _tc4_body,
        grid=(N // R,),
        in_specs=[
            pl.BlockSpec((2, R), lambda i: (0, i)),
            pl.BlockSpec((2, R), lambda i: (0, i)),
            pl.BlockSpec((1, 1), lambda i: (0, 0)),
        ],
        out_specs=[pl.BlockSpec((1, R), lambda i: (0, i))],
        out_shape=[jax.ShapeDtypeStruct((1, N), jnp.float32)],
    )(den3, acc3, b3)


_MESH = plsc.VectorSubcoreMesh(core_axis_name="c", subcore_axis_name="s")
ZR = 125


def _edge_kernel_body(col_split, htab, aux, src_h, dst_h, acc_out, den_out,
                      as_t, ad_t, src_v, dst_v, gidx_v, exv_v, rows_v,
                      zrow_v, zden_v, acc_sp, den_sp, sem):
    c = lax.axis_index("c")
    s = lax.axis_index("s")
    do_den = (c == 0) if col_split else (c >= 0)
    z16f = jnp.zeros((L,), jnp.float32)
    z16i = jnp.zeros((L,), jnp.int32)

    pltpu.sync_copy(aux.at[0], as_t)
    pltpu.sync_copy(aux.at[1], ad_t)

    @pl.loop(0, ZR)
    def _zr(i):
        for j in range(D // L):
            zrow_v[i, pl.ds(L * j, L)] = z16f

    @pl.loop(0, PT // L)
    def _zd(i):
        zden_v[pl.ds(pl.multiple_of(L * i, 8), L)] = z16f

    for k in range(5):
        pltpu.sync_copy(zrow_v, acc_sp.at[pl.ds(s * (N // NS) + k * ZR, ZR)])

    @pl.when(do_den)
    def _():
        pltpu.sync_copy(zden_v, den_sp.at[pl.ds(s * PT, PT)])

    plsc.subcore_barrier()

    if col_split:
        ep = E // NS
        base = s * ep
    else:
        ep = E // (NC * NS)
        base = (c * NS + s) * ep

    @pl.loop(0, ep // CH)
    def _chunk(g):
        eoff = pl.multiple_of(base + g * CH, 8)
        pltpu.sync_copy(src_h.at[pl.ds(eoff, CH)], src_v)
        pltpu.sync_copy(dst_h.at[pl.ds(eoff, CH)], dst_v)
        for j in range(CH // L):
            si = src_v[pl.ds(L * j, L)]
            di = dst_v[pl.ds(L * j, L)]
            av = plsc.load_gather(as_t, [si])
            bv = plsc.load_gather(ad_t, [di])
            e = av + bv
            e = jnp.where(e >= 0.0, e, 0.2 * e)
            exv_v[pl.ds(L * j, L)] = jnp.exp(e)
            if col_split:
                gidx_v[pl.ds(L * j, L)] = si + c * N
        gsrc = gidx_v if col_split else src_v
        pltpu.async_copy(htab.at[gsrc], rows_v, sem).wait()

        @pl.loop(0, CH, unroll=4)
        def _row(i):
            m = plsc.load_gather(exv_v, [z16i + i])
            for j2 in range(D // L):
                rows_v[i, pl.ds(L * j2, L)] = rows_v[i, pl.ds(L * j2, L)] * m

        pltpu.sync_copy(rows_v, acc_sp.at[dst_v], add=True)

        @pl.when(do_den)
        def _():
            pltpu.sync_copy(exv_v, den_sp.at[dst_v], add=True)

    plsc.subcore_barrier()

    for k in range(5):
        pltpu.sync_copy(
            acc_sp.at[pl.ds(s * (N // NS) + k * ZR, ZR)],
            acc_out.at[pl.ds(c * N + s * (N // NS) + k * ZR, ZR)])

    @pl.when(do_den)
    def _():
        if col_split:
            pltpu.sync_copy(den_sp.at[pl.ds(s * PT, PT)],
                            den_out.at[pl.ds(s * PT, PT)])
        else:
            pltpu.sync_copy(den_sp.at[pl.ds(s * PT, PT)],
                            den_out.at[pl.ds(c * NP + s * PT, PT)])


def _make_edge_kernel(col_split):
    den_len = NP if col_split else NC * NP
    return pl.kernel(
        functools.partial(_edge_kernel_body, col_split),
        out_type=[
            jax.ShapeDtypeStruct((NC * N, D), jnp.float32),
            jax.ShapeDtypeStruct((den_len,), jnp.float32),
        ],
        mesh=_MESH,
        scratch_types=[
            pltpu.VMEM((N,), jnp.float32),
            pltpu.VMEM((N,), jnp.float32),
            pltpu.VMEM((CH,), jnp.int32),
            pltpu.VMEM((CH,), jnp.int32),
            pltpu.VMEM((CH,), jnp.int32),
            pltpu.VMEM((CH,), jnp.float32),
            pltpu.VMEM((CH, D), jnp.float32),
            pltpu.VMEM((ZR, D), jnp.float32),
            pltpu.VMEM((PT,), jnp.float32),
            pltpu.VMEM_SHARED((N, D), jnp.float32),
            pltpu.VMEM_SHARED((NP,), jnp.float32),
            pltpu.SemaphoreType.DMA,
        ],
    )


_sc_l1 = _make_edge_kernel(col_split=True)
_sc_l2 = _make_edge_kernel(col_split=False)


def _sc_l3_body(aux, src_h, dst_h, den_out, acc_out,
                as_t, ad_t, h_t, src_v, dst_v, exv_v, pv_v, zden_v,
                den_sp, acc_sp):
    c = lax.axis_index("c")
    s = lax.axis_index("s")
    z16f = jnp.zeros((L,), jnp.float32)

    pltpu.sync_copy(aux.at[0], h_t)
    pltpu.sync_copy(aux.at[1], as_t)
    pltpu.sync_copy(aux.at[2], ad_t)

    @pl.loop(0, PT // L)
    def _zd(i):
        zden_v[pl.ds(pl.multiple_of(L * i, 8), L)] = z16f

    pltpu.sync_copy(zden_v, den_sp.at[pl.ds(s * PT, PT)])
    pltpu.sync_copy(zden_v, acc_sp.at[pl.ds(s * PT, PT)])
    plsc.subcore_barrier()

    ep = E // (NC * NS)
    base = (c * NS + s) * ep

    @pl.loop(0, ep // CH)
    def _chunk(g):
        eoff = pl.multiple_of(base + g * CH, 8)
        pltpu.sync_copy(src_h.at[pl.ds(eoff, CH)], src_v)
        pltpu.sync_copy(dst_h.at[pl.ds(eoff, CH)], dst_v)
        for j in range(CH // L):
            si = src_v[pl.ds(L * j, L)]
            di = dst_v[pl.ds(L * j, L)]
            av = plsc.load_gather(as_t, [si])
            bv = plsc.load_gather(ad_t, [di])
            hv = plsc.load_gather(h_t, [si])
            e = av + bv
            e = jnp.where(e >= 0.0, e, 0.2 * e)
            ex = jnp.exp(e)
            exv_v[pl.ds(L * j, L)] = ex
            pv_v[pl.ds(L * j, L)] = ex * hv
        pltpu.sync_copy(exv_v, den_sp.at[dst_v], add=True)
        pltpu.sync_copy(pv_v, acc_sp.at[dst_v], add=True)

    plsc.subcore_barrier()
    pltpu.sync_copy(den_sp.at[pl.ds(s * PT, PT)],
                    den_out.at[pl.ds(c * NP + s * PT, PT)])
    pltpu.sync_copy(acc_sp.at[pl.ds(s * PT, PT)],
                    acc_out.at[pl.ds(c * NP + s * PT, PT)])


_sc_l3 = pl.kernel(
    _sc_l3_body,
    out_type=[
        jax.ShapeDtypeStruct((NC * NP,), jnp.float32),
        jax.ShapeDtypeStruct((NC * NP,), jnp.float32),
    ],
    mesh=_MESH,
    scratch_types=[
        pltpu.VMEM((N,), jnp.float32),
        pltpu.VMEM((N,), jnp.float32),
        pltpu.VMEM((N,), jnp.float32),
        pltpu.VMEM((CH,), jnp.int32),
        pltpu.VMEM((CH,), jnp.int32),
        pltpu.VMEM((CH,), jnp.float32),
        pltpu.VMEM((CH,), jnp.float32),
        pltpu.VMEM((PT,), jnp.float32),
        pltpu.VMEM_SHARED((NP,), jnp.float32),
        pltpu.VMEM_SHARED((NP,), jnp.float32),
    ],
)


def kernel(x_i, x_j, edge_index, W1, a_src1, a_dst1, b1,
           W2, a_src2, a_dst2, b2, W3, a_src3, a_dst3, b3):
    src = edge_index[0]
    dst = edge_index[1]

    ht, hb, aux1 = _tc1(x_i, x_j, W1,
                        a_src1.reshape(1, -1), a_dst1.reshape(1, -1))
    htab1 = jnp.concatenate([ht, hb], axis=0)
    acc1, den1 = _sc_l1(htab1, aux1, src, dst)

    h2, aux2 = _tc2(acc1, den1[:N].reshape(1, N), b1.reshape(1, -1),
                    W2, a_src2.reshape(1, -1), a_dst2.reshape(1, -1))
    acc2, den2 = _sc_l2(h2, aux2, src, dst)

    aux3 = _tc3(acc2, den2.reshape(NC, NP)[:, :N], b2.reshape(1, -1),
                W3.reshape(1, D), a_src3.reshape(1, 1), a_dst3.reshape(1, 1))[0]
    den3, acc3 = _sc_l3(aux3, src, dst)

    out = _tc4(den3.reshape(NC, NP)[:, :N], acc3.reshape(NC, NP)[:, :N],
               b3.reshape(1, 1))[0]
    return out.reshape(N, 1)


# speedup vs baseline: 18.7918x; 18.7918x over previous
import functools

import jax
import jax.numpy as jnp
from jax import lax
from jax.experimental import pallas as pl
from jax.experimental.pallas import tpu as pltpu
from jax.experimental.pallas import tpu_sc as plsc

N = 10000
E = 320000
D = 128

NC = 2
NS = 16
L = 16
CH = 80

NP = 10240
R = 1024
PT = NP // NS
ZR = 128

_HIGH = lax.Precision.HIGHEST


def _dot(a, b):
    return jnp.dot(a, b, precision=_HIGH, preferred_element_type=jnp.float32)


def _tc1_body(xi, xj, w1, asr, adr, ht, hb, aux):
    h = _dot(xi[...], w1[:D, :]) + _dot(xj[...], w1[D:, :])
    ht[...] = h[:, :D]
    hb[...] = h[:, D:]
    a = jnp.sum(h * asr[...], axis=1)
    b = jnp.sum(h * adr[...], axis=1)
    aux[...] = jnp.concatenate(
        [a[None, :], b[None, :], jnp.zeros((6, R), jnp.float32)], axis=0)


def _tc1(xi, xj, w1, asr, adr):
    return pl.pallas_call(
        _tc1_body,
        grid=(NP // R,),
        in_specs=[
            pl.BlockSpec((R, D), lambda i: (i, 0)),
            pl.BlockSpec((R, D), lambda i: (i, 0)),
            pl.BlockSpec((2 * D, 2 * D), lambda i: (0, 0)),
            pl.BlockSpec((1, 2 * D), lambda i: (0, 0)),
            pl.BlockSpec((1, 2 * D), lambda i: (0, 0)),
        ],
        out_specs=[
            pl.BlockSpec((R, D), lambda i: (i, 0)),
            pl.BlockSpec((R, D), lambda i: (i, 0)),
            pl.BlockSpec((8, R), lambda i: (0, i)),
        ],
        out_shape=[
            jax.ShapeDtypeStruct((NP, D), jnp.float32),
            jax.ShapeDtypeStruct((NP, D), jnp.float32),
            jax.ShapeDtypeStruct((8, NP), jnp.float32),
        ],
    )(xi, xj, w1, asr, adr)


def _tc2_body(accA, accB, den, b1, w2, asr, adr, h2, aux):
    d = den[0, :][:, None] + 1e-16
    x2a = jax.nn.relu(accA[...] / d + b1[0, :D])
    x2b = jax.nn.relu(accB[...] / d + b1[0, D:])
    h = _dot(x2a, w2[:D, :]) + _dot(x2b, w2[D:, :])
    h2[...] = h
    a = jnp.sum(h * asr[...], axis=1)
    b = jnp.sum(h * adr[...], axis=1)
    aux[...] = jnp.concatenate(
        [a[None, :], b[None, :], jnp.zeros((6, R), jnp.float32)], axis=0)


def _tc2(acc1, den1, b1, w2, asr, adr):
    return pl.pallas_call(
        _tc2_body,
        grid=(NP // R,),
        in_specs=[
            pl.BlockSpec((R, D), lambda i: (i, 0)),
            pl.BlockSpec((R, D), lambda i: (NP // R + i, 0)),
            pl.BlockSpec((1, R), lambda i: (0, i)),
            pl.BlockSpec((1, 2 * D), lambda i: (0, 0)),
            pl.BlockSpec((2 * D, D), lambda i: (0, 0)),
            pl.BlockSpec((1, D), lambda i: (0, 0)),
            pl.BlockSpec((1, D), lambda i: (0, 0)),
        ],
        out_specs=[
            pl.BlockSpec((R, D), lambda i: (i, 0)),
            pl.BlockSpec((8, R), lambda i: (0, i)),
        ],
        out_shape=[
            jax.ShapeDtypeStruct((NP, D), jnp.float32),
            jax.ShapeDtypeStruct((8, NP), jnp.float32),
        ],
    )(acc1, acc1, den1, b1, w2, asr, adr)


def _tc3_body(accA, accB, den, b2, w3, as3, ad3, aux):
    d = (den[0, :] + den[1, :])[:, None] + 1e-16
    x3 = jax.nn.relu((accA[...] + accB[...]) / d + b2[0, :])
    h3 = jnp.sum(x3 * w3[...], axis=1)
    aux[...] = jnp.concatenate(
        [h3[None, :], (h3 * as3[0, 0])[None, :], (h3 * ad3[0, 0])[None, :],
         jnp.zeros((5, R), jnp.float32)], axis=0)


def _tc3(acc2, den2, b2, w3, as3, ad3):
    return pl.pallas_call(
        _tc3_body,
        grid=(NP // R,),
        in_specs=[
            pl.BlockSpec((R, D), lambda i: (i, 0)),
            pl.BlockSpec((R, D), lambda i: (NP // R + i, 0)),
            pl.BlockSpec((2, R), lambda i: (0, i)),
            pl.BlockSpec((1, D), lambda i: (0, 0)),
            pl.BlockSpec((1, D), lambda i: (0, 0)),
            pl.BlockSpec((1, 1), lambda i: (0, 0)),
            pl.BlockSpec((1, 1), lambda i: (0, 0)),
        ],
        out_specs=[pl.BlockSpec((8, R), lambda i: (0, i))],
        out_shape=[jax.ShapeDtypeStruct((8, NP), jnp.float32)],
    )(acc2, acc2, den2, b2, w3, as3, ad3)


def _tc4_body(den, acc, b3, out):
    d = den[0, :] + den[1, :] + 1e-16
    a = acc[0, :] + acc[1, :]
    out[...] = jax.nn.sigmoid(a / d + b3[0, 0])[None, :]


def _tc4(den3, acc3, b3):
    return pl.pallas_call(
        _tc4_body,
        grid=(NP // R,),
        in_specs=[
            pl.BlockSpec((2, R), lambda i: (0, i)),
            pl.BlockSpec((2, R), lambda i: (0, i)),
            pl.BlockSpec((1, 1), lambda i: (0, 0)),
        ],
        out_specs=[pl.BlockSpec((1, R), lambda i: (0, i))],
        out_shape=[jax.ShapeDtypeStruct((1, NP), jnp.float32)],
    )(den3, acc3, b3)


_MESH = plsc.VectorSubcoreMesh(core_axis_name="c", subcore_axis_name="s")
_SC_PARAMS = pltpu.CompilerParams(needs_layout_passes=False)


def _edge_kernel_body(col_split, htab, aux, src_h, dst_h, acc_out, den_out,
                      as_t, ad_t, src_v, dst_v, gidx_v, exv_v, rows_v,
                      zrow_v, zden_v, acc_sp, den_sp, sem):
    c = lax.axis_index("c")
    s = lax.axis_index("s")
    do_den = (c == 0) if col_split else (c >= 0)
    z16f = jnp.zeros((L,), jnp.float32)
    z16i = jnp.zeros((L,), jnp.int32)

    pltpu.sync_copy(aux.at[0], as_t)
    pltpu.sync_copy(aux.at[1], ad_t)

    @pl.loop(0, ZR)
    def _zr(i):
        for j in range(D // L):
            zrow_v[i, pl.ds(L * j, L)] = z16f

    @pl.loop(0, PT // L)
    def _zd(i):
        zden_v[pl.ds(pl.multiple_of(L * i, 8), L)] = z16f

    for k in range(PT // ZR):
        pltpu.sync_copy(zrow_v, acc_sp.at[pl.ds(s * PT + k * ZR, ZR)])

    @pl.when(do_den)
    def _():
        pltpu.sync_copy(zden_v, den_sp.at[pl.ds(s * PT, PT)])

    plsc.subcore_barrier()

    if col_split:
        ep = E // NS
        base = s * ep
    else:
        ep = E // (NC * NS)
        base = (c * NS + s) * ep

    @pl.loop(0, ep // CH)
    def _chunk(g):
        eoff = pl.multiple_of(base + g * CH, 8)
        pltpu.sync_copy(src_h.at[pl.ds(eoff, CH)], src_v)
        pltpu.sync_copy(dst_h.at[pl.ds(eoff, CH)], dst_v)
        for j in range(CH // L):
            si = src_v[pl.ds(L * j, L)]
            di = dst_v[pl.ds(L * j, L)]
            av = plsc.load_gather(as_t, [si])
            bv = plsc.load_gather(ad_t, [di])
            e = av + bv
            e = jnp.where(e >= 0.0, e, 0.2 * e)
            exv_v[pl.ds(L * j, L)] = jnp.exp(e)
            if col_split:
                gidx_v[pl.ds(L * j, L)] = si + c * NP
        gsrc = gidx_v if col_split else src_v
        pltpu.async_copy(htab.at[gsrc], rows_v, sem).wait()

        @pl.loop(0, CH, unroll=4)
        def _row(i):
            m = plsc.load_gather(exv_v, [z16i + i])
            for j2 in range(D // L):
                rows_v[i, pl.ds(L * j2, L)] = rows_v[i, pl.ds(L * j2, L)] * m

        pltpu.sync_copy(rows_v, acc_sp.at[dst_v], add=True)

        @pl.when(do_den)
        def _():
            pltpu.sync_copy(exv_v, den_sp.at[dst_v], add=True)

    plsc.subcore_barrier()

    for k in range(PT // ZR):
        pltpu.sync_copy(
            acc_sp.at[pl.ds(s * PT + k * ZR, ZR)],
            acc_out.at[pl.ds(c * NP + s * PT + k * ZR, ZR)])

    @pl.when(do_den)
    def _():
        if col_split:
            pltpu.sync_copy(den_sp.at[pl.ds(s * PT, PT)],
                            den_out.at[pl.ds(s * PT, PT)])
        else:
            pltpu.sync_copy(den_sp.at[pl.ds(s * PT, PT)],
                            den_out.at[pl.ds(c * NP + s * PT, PT)])


def _make_edge_kernel(col_split):
    den_len = NP if col_split else NC * NP
    return pl.kernel(
        functools.partial(_edge_kernel_body, col_split),
        out_type=[
            jax.ShapeDtypeStruct((NC * NP, D), jnp.float32),
            jax.ShapeDtypeStruct((den_len,), jnp.float32),
        ],
        mesh=_MESH,
        compiler_params=_SC_PARAMS,
        scratch_types=[
            pltpu.VMEM((NP,), jnp.float32),
            pltpu.VMEM((NP,), jnp.float32),
            pltpu.VMEM((CH,), jnp.int32),
            pltpu.VMEM((CH,), jnp.int32),
            pltpu.VMEM((CH,), jnp.int32),
            pltpu.VMEM((CH,), jnp.float32),
            pltpu.VMEM((CH, D), jnp.float32),
            pltpu.VMEM((ZR, D), jnp.float32),
            pltpu.VMEM((PT,), jnp.float32),
            pltpu.VMEM_SHARED((NP, D), jnp.float32),
            pltpu.VMEM_SHARED((NP,), jnp.float32),
            pltpu.SemaphoreType.DMA,
        ],
    )


_sc_l1 = _make_edge_kernel(col_split=True)
_sc_l2 = _make_edge_kernel(col_split=False)


def _sc_l3_body(aux, src_h, dst_h, den_out, acc_out,
                as_t, ad_t, h_t, src_v, dst_v, exv_v, pv_v, zden_v,
                den_sp, acc_sp):
    c = lax.axis_index("c")
    s = lax.axis_index("s")
    z16f = jnp.zeros((L,), jnp.float32)

    pltpu.sync_copy(aux.at[0], h_t)
    pltpu.sync_copy(aux.at[1], as_t)
    pltpu.sync_copy(aux.at[2], ad_t)

    @pl.loop(0, PT // L)
    def _zd(i):
        zden_v[pl.ds(pl.multiple_of(L * i, 8), L)] = z16f

    pltpu.sync_copy(zden_v, den_sp.at[pl.ds(s * PT, PT)])
    pltpu.sync_copy(zden_v, acc_sp.at[pl.ds(s * PT, PT)])
    plsc.subcore_barrier()

    ep = E // (NC * NS)
    base = (c * NS + s) * ep

    @pl.loop(0, ep // CH)
    def _chunk(g):
        eoff = pl.multiple_of(base + g * CH, 8)
        pltpu.sync_copy(src_h.at[pl.ds(eoff, CH)], src_v)
        pltpu.sync_copy(dst_h.at[pl.ds(eoff, CH)], dst_v)
        for j in range(CH // L):
            si = src_v[pl.ds(L * j, L)]
            di = dst_v[pl.ds(L * j, L)]
            av = plsc.load_gather(as_t, [si])
            bv = plsc.load_gather(ad_t, [di])
            hv = plsc.load_gather(h_t, [si])
            e = av + bv
            e = jnp.where(e >= 0.0, e, 0.2 * e)
            ex = jnp.exp(e)
            exv_v[pl.ds(L * j, L)] = ex
            pv_v[pl.ds(L * j, L)] = ex * hv
        pltpu.sync_copy(exv_v, den_sp.at[dst_v], add=True)
        pltpu.sync_copy(pv_v, acc_sp.at[dst_v], add=True)

    plsc.subcore_barrier()
    pltpu.sync_copy(den_sp.at[pl.ds(s * PT, PT)],
                    den_out.at[pl.ds(c * NP + s * PT, PT)])
    pltpu.sync_copy(acc_sp.at[pl.ds(s * PT, PT)],
                    acc_out.at[pl.ds(c * NP + s * PT, PT)])


_sc_l3 = pl.kernel(
    _sc_l3_body,
    out_type=[
        jax.ShapeDtypeStruct((NC * NP,), jnp.float32),
        jax.ShapeDtypeStruct((NC * NP,), jnp.float32),
    ],
    mesh=_MESH,
    compiler_params=_SC_PARAMS,
    scratch_types=[
        pltpu.VMEM((NP,), jnp.float32),
        pltpu.VMEM((NP,), jnp.float32),
        pltpu.VMEM((NP,), jnp.float32),
        pltpu.VMEM((CH,), jnp.int32),
        pltpu.VMEM((CH,), jnp.int32),
        pltpu.VMEM((CH,), jnp.float32),
        pltpu.VMEM((CH,), jnp.float32),
        pltpu.VMEM((PT,), jnp.float32),
        pltpu.VMEM_SHARED((NP,), jnp.float32),
        pltpu.VMEM_SHARED((NP,), jnp.float32),
    ],
)


def kernel(x_i, x_j, edge_index, W1, a_src1, a_dst1, b1,
           W2, a_src2, a_dst2, b2, W3, a_src3, a_dst3, b3):
    src = edge_index[0]
    dst = edge_index[1]
    pad = ((0, NP - N), (0, 0))
    xi = jnp.pad(x_i, pad)
    xj = jnp.pad(x_j, pad)

    ht, hb, aux1 = _tc1(xi, xj, W1,
                        a_src1.reshape(1, -1), a_dst1.reshape(1, -1))
    htab1 = jnp.concatenate([ht, hb], axis=0)
    acc1, den1 = _sc_l1(htab1, aux1, src, dst)

    h2, aux2 = _tc2(acc1, den1.reshape(1, NP), b1.reshape(1, -1),
                    W2, a_src2.reshape(1, -1), a_dst2.reshape(1, -1))
    acc2, den2 = _sc_l2(h2, aux2, src, dst)

    aux3 = _tc3(acc2, den2.reshape(NC, NP), b2.reshape(1, -1),
                W3.reshape(1, D), a_src3.reshape(1, 1), a_dst3.reshape(1, 1))[0]
    den3, acc3 = _sc_l3(aux3, src, dst)

    out = _tc4(den3.reshape(NC, NP), acc3.reshape(NC, NP),
               b3.reshape(1, 1))[0]
    return out[0, :N].reshape(N, 1)


# speedup vs baseline: 19.2003x; 1.0217x over previous
import functools

import jax
import jax.numpy as jnp
from jax import lax
from jax.experimental import pallas as pl
from jax.experimental.pallas import tpu as pltpu
from jax.experimental.pallas import tpu_sc as plsc

N = 10000
E = 320000
D = 128

NC = 2
NS = 16
L = 16
CH = 80

NP = 10240
R = 1024
PT = NP // NS
ZR = 64
D2 = 64

_HIGH = lax.Precision.HIGHEST


def _dot(a, b):
    return jnp.dot(a, b, precision=_HIGH, preferred_element_type=jnp.float32)


def _tc1_body(xi, xj, w1, asr, adr, ht, hb, aux):
    h = _dot(xi[...], w1[:D, :]) + _dot(xj[...], w1[D:, :])
    ht[...] = h[:, :D]
    hb[...] = h[:, D:]
    a = jnp.sum(h * asr[...], axis=1)
    b = jnp.sum(h * adr[...], axis=1)
    aux[...] = jnp.concatenate(
        [a[None, :], b[None, :], jnp.zeros((6, R), jnp.float32)], axis=0)


def _tc1(xi, xj, w1, asr, adr):
    return pl.pallas_call(
        _tc1_body,
        grid=(NP // R,),
        in_specs=[
            pl.BlockSpec((R, D), lambda i: (i, 0)),
            pl.BlockSpec((R, D), lambda i: (i, 0)),
            pl.BlockSpec((2 * D, 2 * D), lambda i: (0, 0)),
            pl.BlockSpec((1, 2 * D), lambda i: (0, 0)),
            pl.BlockSpec((1, 2 * D), lambda i: (0, 0)),
        ],
        out_specs=[
            pl.BlockSpec((R, D), lambda i: (i, 0)),
            pl.BlockSpec((R, D), lambda i: (i, 0)),
            pl.BlockSpec((8, R), lambda i: (0, i)),
        ],
        out_shape=[
            jax.ShapeDtypeStruct((NP, D), jnp.float32),
            jax.ShapeDtypeStruct((NP, D), jnp.float32),
            jax.ShapeDtypeStruct((8, NP), jnp.float32),
        ],
    )(xi, xj, w1, asr, adr)


def _tc2_body(accA, accB, den, b1, w2, asr, adr, h2, aux):
    d = den[0, :][:, None] + 1e-16
    x2a = jax.nn.relu(accA[...] / d + b1[0, :D])
    x2b = jax.nn.relu(accB[...] / d + b1[0, D:])
    h = _dot(x2a, w2[:D, :]) + _dot(x2b, w2[D:, :])
    h2[...] = h
    a = jnp.sum(h * asr[...], axis=1)
    b = jnp.sum(h * adr[...], axis=1)
    aux[...] = jnp.concatenate(
        [a[None, :], b[None, :], jnp.zeros((6, R), jnp.float32)], axis=0)


def _tc2(acc1, den1, b1, w2, asr, adr):
    return pl.pallas_call(
        _tc2_body,
        grid=(NP // R,),
        in_specs=[
            pl.BlockSpec((R, D), lambda i: (i, 0)),
            pl.BlockSpec((R, D), lambda i: (NP // R + i, 0)),
            pl.BlockSpec((1, R), lambda i: (0, i)),
            pl.BlockSpec((1, 2 * D), lambda i: (0, 0)),
            pl.BlockSpec((2 * D, D), lambda i: (0, 0)),
            pl.BlockSpec((1, D), lambda i: (0, 0)),
            pl.BlockSpec((1, D), lambda i: (0, 0)),
        ],
        out_specs=[
            pl.BlockSpec((R, D), lambda i: (i, 0)),
            pl.BlockSpec((8, R), lambda i: (0, i)),
        ],
        out_shape=[
            jax.ShapeDtypeStruct((NP, D), jnp.float32),
            jax.ShapeDtypeStruct((8, NP), jnp.float32),
        ],
    )(acc1, acc1, den1, b1, w2, asr, adr)


def _tc3_body(accA, accB, den, b2, w3, as3, ad3, aux):
    d = (den[0, :] + den[1, :])[:, None] + 1e-16
    x3 = jax.nn.relu((accA[...] + accB[...]) / d + b2[0, :])
    h3 = jnp.sum(x3 * w3[...], axis=1)
    aux[...] = jnp.concatenate(
        [h3[None, :], (h3 * as3[0, 0])[None, :], (h3 * ad3[0, 0])[None, :],
         jnp.zeros((5, R), jnp.float32)], axis=0)


def _tc3(acc2, den2, b2, w3, as3, ad3):
    return pl.pallas_call(
        _tc3_body,
        grid=(NP // R,),
        in_specs=[
            pl.BlockSpec((R, D), lambda i: (i, 0)),
            pl.BlockSpec((R, D), lambda i: (NP // R + i, 0)),
            pl.BlockSpec((2, R), lambda i: (0, i)),
            pl.BlockSpec((1, D), lambda i: (0, 0)),
            pl.BlockSpec((1, D), lambda i: (0, 0)),
            pl.BlockSpec((1, 1), lambda i: (0, 0)),
            pl.BlockSpec((1, 1), lambda i: (0, 0)),
        ],
        out_specs=[pl.BlockSpec((8, R), lambda i: (0, i))],
        out_shape=[jax.ShapeDtypeStruct((8, NP), jnp.float32)],
    )(acc2, acc2, den2, b2, w3, as3, ad3)


def _tc4_body(den, acc, b3, out):
    d = den[0, :] + den[1, :] + 1e-16
    a = acc[0, :] + acc[1, :]
    out[...] = jax.nn.sigmoid(a / d + b3[0, 0])[None, :]


def _tc4(den3, acc3, b3):
    return pl.pallas_call(
        _tc4_body,
        grid=(NP // R,),
        in_specs=[
            pl.BlockSpec((2, R), lambda i: (0, i)),
            pl.BlockSpec((2, R), lambda i: (0, i)),
            pl.BlockSpec((1, 1), lambda i: (0, 0)),
        ],
        out_specs=[pl.BlockSpec((1, R), lambda i: (0, i))],
        out_shape=[jax.ShapeDtypeStruct((1, NP), jnp.float32)],
    )(den3, acc3, b3)


_MESH = plsc.VectorSubcoreMesh(core_axis_name="c", subcore_axis_name="s")
_SC_PARAMS = pltpu.CompilerParams(needs_layout_passes=False)
NB = 5


NH = NP // 2
PT2 = NH // NS


def _edge_body(col_split, htab, aux, src_h, dst_h, acc_out, den_out,
               as_t, ad_t, sidx, rows, gidx, exv, dst_v, dstl,
               zrow_v, zden_v, acc_sp, den_sp, semi, semg):
    c = lax.axis_index("c")
    s = lax.axis_index("s")
    den_cond = (c == 0) if col_split else (c >= 0)
    z16f = jnp.zeros((L,), jnp.float32)
    z16i = jnp.zeros((L,), jnp.int32)

    if col_split:
        ep = E // NS
        base = s * ep
    else:
        ep = E // (NC * NS)
        base = (c * NS + s) * ep
    nch = ep // CH

    pltpu.sync_copy(aux.at[0], as_t)
    pltpu.sync_copy(aux.at[1], ad_t)

    @pl.loop(0, ZR)
    def _zr(i):
        for j in range(D // L):
            zrow_v[i, pl.ds(L * j, L)] = z16f

    @pl.loop(0, PT // L)
    def _zd(i):
        zden_v[pl.ds(pl.multiple_of(L * i, 8), L)] = z16f

    @pl.when(den_cond)
    def _():
        pltpu.sync_copy(zden_v, den_sp.at[pl.ds(s * PT, PT)])

    def issue_idx(b, g):
        eoff = pl.multiple_of(base + g * CH, 8)
        pltpu.async_copy(src_h.at[pl.ds(eoff, CH)], sidx[b], semi[b])
        pltpu.async_copy(dst_h.at[pl.ds(eoff, CH)], dst_v[b], semi[b])

    for p_half in range(2):
        lo = p_half * NH

        for k in range(PT2 // ZR):
            pltpu.sync_copy(zrow_v, acc_sp.at[pl.ds(s * PT2 + k * ZR, ZR)])

        plsc.subcore_barrier()

        def prep(b):
            pltpu.make_async_copy(src_h.at[pl.ds(0, CH)], sidx[b],
                                  semi[b]).wait()
            pltpu.make_async_copy(dst_h.at[pl.ds(0, CH)], dst_v[b],
                                  semi[b]).wait()
            for j in range(CH // L):
                si = sidx[b][pl.ds(L * j, L)]
                di = dst_v[b][pl.ds(L * j, L)]
                av = plsc.load_gather(as_t, [si])
                bv = plsc.load_gather(ad_t, [di])
                e = av + bv
                e = jnp.where(e >= 0.0, e, 0.2 * e)
                exv[b][pl.ds(L * j, L)] = jnp.exp(e)
                gidx[b][pl.ds(L * j, L)] = (si + c * NP) if col_split else si
                dl = di - lo
                dstl[b][pl.ds(L * j, L)] = jnp.where(
                    (dl >= 0) & (dl < NH), dl, NH)
            pltpu.async_copy(htab.at[gidx[b]], rows[b], semg[b])

        def finish(b):
            pltpu.make_async_copy(htab.at[gidx[b]], rows[b],
                                  semg[b]).wait()

            @pl.loop(0, CH, unroll=4)
            def _row(i):
                m = plsc.load_gather(exv[b], [z16i + i])
                for j2 in range(D // L):
                    rows[b][i, pl.ds(L * j2, L)] = (
                        rows[b][i, pl.ds(L * j2, L)] * m)

            pltpu.sync_copy(rows[b], acc_sp.at[dstl[b]], add=True)
            if p_half == 0:
                @pl.when(den_cond)
                def _():
                    pltpu.sync_copy(exv[b], den_sp.at[dst_v[b]], add=True)

        for b in range(NB):
            issue_idx(b, b)
        for b in range(NB):
            prep(b)

        @pl.loop(0, nch // NB - 1)
        def _grp(p):
            g0 = p * NB
            for b in range(NB):
                finish(b)
                issue_idx(b, g0 + NB + b)
            for b in range(NB):
                prep(b)

        for b in range(NB):
            finish(b)

        plsc.subcore_barrier()

        for k in range(PT2 // ZR):
            pltpu.sync_copy(
                acc_sp.at[pl.ds(s * PT2 + k * ZR, ZR)],
                acc_out.at[pl.ds(c * NP + lo + s * PT2 + k * ZR, ZR)])

        if p_half == 0:
            @pl.when(den_cond)
            def _():
                if col_split:
                    pltpu.sync_copy(den_sp.at[pl.ds(s * PT, PT)],
                                    den_out.at[pl.ds(s * PT, PT)])
                else:
                    pltpu.sync_copy(den_sp.at[pl.ds(s * PT, PT)],
                                    den_out.at[pl.ds(c * NP + s * PT, PT)])


def _make_edge(col_split):
    den_len = NP if col_split else NC * NP
    return pl.kernel(
        functools.partial(_edge_body, col_split),
        out_type=[
            jax.ShapeDtypeStruct((NC * NP, D), jnp.float32),
            jax.ShapeDtypeStruct((den_len,), jnp.float32),
        ],
        mesh=_MESH,
        compiler_params=_SC_PARAMS,
        scratch_types=[
            pltpu.VMEM((NP,), jnp.float32),
            pltpu.VMEM((NP,), jnp.float32),
            tuple(pltpu.VMEM((CH,), jnp.int32) for _ in range(NB)),
            tuple(pltpu.VMEM((CH, D), jnp.float32) for _ in range(NB)),
            tuple(pltpu.VMEM((CH,), jnp.int32) for _ in range(NB)),
            tuple(pltpu.VMEM((CH,), jnp.float32) for _ in range(NB)),
            tuple(pltpu.VMEM((CH,), jnp.int32) for _ in range(NB)),
            tuple(pltpu.VMEM((CH,), jnp.int32) for _ in range(NB)),
            pltpu.VMEM((ZR, D), jnp.float32),
            pltpu.VMEM((PT,), jnp.float32),
            pltpu.VMEM_SHARED((NH + 8, D), jnp.float32),
            pltpu.VMEM_SHARED((NP,), jnp.float32),
            tuple(pltpu.SemaphoreType.DMA for _ in range(NB)),
            tuple(pltpu.SemaphoreType.DMA for _ in range(NB)),
        ],
    )


_sc_l1 = _make_edge(col_split=True)
_sc_l2 = _make_edge(col_split=False)


_L3_EP = E // (NC * NS)
_L3_NCH = _L3_EP // CH


def _sc_l3_body(aux, src_h, dst3d_h, den_out, acc_out,
                as_t, ad_t, h_t, srcb, dstb, exv, pv, dst_v, zden_v,
                den_sp, acc_sp, semd, sema):
    c = lax.axis_index("c")
    s = lax.axis_index("s")
    z16f = jnp.zeros((L,), jnp.float32)

    base = (c * NS + s) * _L3_EP

    pltpu.sync_copy(aux.at[0], h_t)
    pltpu.sync_copy(aux.at[1], as_t)
    pltpu.sync_copy(aux.at[2], ad_t)
    pltpu.sync_copy(src_h.at[pl.ds(pl.multiple_of(base, 8), _L3_EP)], srcb)
    pltpu.sync_copy(dst3d_h.at[c * NS + s], dstb)

    @pl.loop(0, PT // L)
    def _zd(i):
        zden_v[pl.ds(pl.multiple_of(L * i, 8), L)] = z16f

    pltpu.sync_copy(zden_v, den_sp.at[pl.ds(s * PT, PT)])
    pltpu.sync_copy(zden_v, acc_sp.at[pl.ds(s * PT, PT)])
    plsc.subcore_barrier()

    def prep3(b, g):
        for j in range(CH // L):
            si = srcb[pl.ds(g * CH + L * j, L)]
            di = dstb[g, pl.ds(L * j, L)]
            av = plsc.load_gather(as_t, [si])
            bv = plsc.load_gather(ad_t, [di])
            hv = plsc.load_gather(h_t, [si])
            e = av + bv
            e = jnp.where(e >= 0.0, e, 0.2 * e)
            ex = jnp.exp(e)
            exv[b][pl.ds(L * j, L)] = ex
            pv[b][pl.ds(L * j, L)] = ex * hv
            dst_v[b][pl.ds(L * j, L)] = di
        pltpu.async_copy(exv[b], den_sp.at[dst_v[b]], semd[b], add=True)
        pltpu.async_copy(pv[b], acc_sp.at[dst_v[b]], sema[b], add=True)

    def drain3(b):
        pltpu.make_async_copy(exv[b], den_sp.at[dst_v[b]], semd[b]).wait()
        pltpu.make_async_copy(pv[b], acc_sp.at[dst_v[b]], sema[b]).wait()

    for b in range(NB):
        prep3(b, b)

    @pl.loop(0, _L3_NCH // NB - 1)
    def _grp(p):
        g0 = p * NB
        for b in range(NB):
            drain3(b)
            prep3(b, g0 + NB + b)

    for b in range(NB):
        drain3(b)

    plsc.subcore_barrier()
    pltpu.sync_copy(den_sp.at[pl.ds(s * PT, PT)],
                    den_out.at[pl.ds(c * NP + s * PT, PT)])
    pltpu.sync_copy(acc_sp.at[pl.ds(s * PT, PT)],
                    acc_out.at[pl.ds(c * NP + s * PT, PT)])


_sc_l3 = pl.kernel(
    _sc_l3_body,
    out_type=[
        jax.ShapeDtypeStruct((NC * NP,), jnp.float32),
        jax.ShapeDtypeStruct((NC * NP,), jnp.float32),
    ],
    mesh=_MESH,
    compiler_params=_SC_PARAMS,
    scratch_types=[
        pltpu.VMEM((NP,), jnp.float32),
        pltpu.VMEM((NP,), jnp.float32),
        pltpu.VMEM((NP,), jnp.float32),
        pltpu.VMEM((_L3_EP,), jnp.int32),
        pltpu.VMEM((_L3_NCH, CH), jnp.int32),
        tuple(pltpu.VMEM((CH,), jnp.float32) for _ in range(NB)),
        tuple(pltpu.VMEM((CH,), jnp.float32) for _ in range(NB)),
        tuple(pltpu.VMEM((CH,), jnp.int32) for _ in range(NB)),
        pltpu.VMEM((PT,), jnp.float32),
        pltpu.VMEM_SHARED((NP,), jnp.float32),
        pltpu.VMEM_SHARED((NP,), jnp.float32),
        tuple(pltpu.SemaphoreType.DMA for _ in range(NB)),
        tuple(pltpu.SemaphoreType.DMA for _ in range(NB)),
    ],
)


def kernel(x_i, x_j, edge_index, W1, a_src1, a_dst1, b1,
           W2, a_src2, a_dst2, b2, W3, a_src3, a_dst3, b3):
    src = edge_index[0]
    dst = edge_index[1]
    dst3d_32 = dst.reshape(NC * NS, -1, CH)
    pad = ((0, NP - N), (0, 0))
    xi = jnp.pad(x_i, pad)
    xj = jnp.pad(x_j, pad)

    ht, hb, aux1 = _tc1(xi, xj, W1,
                        a_src1.reshape(1, -1), a_dst1.reshape(1, -1))
    htab1 = jnp.concatenate([ht, hb], axis=0)
    acc1, den1 = _sc_l1(htab1, aux1, src, dst)

    h2, aux2 = _tc2(acc1, den1.reshape(1, NP), b1.reshape(1, -1),
                    W2, a_src2.reshape(1, -1), a_dst2.reshape(1, -1))
    acc2, den2 = _sc_l2(h2, aux2, src, dst)

    aux3 = _tc3(acc2, den2.reshape(NC, NP), b2.reshape(1, -1),
                W3.reshape(1, D), a_src3.reshape(1, 1), a_dst3.reshape(1, 1))[0]
    den3, acc3 = _sc_l3(aux3, src, dst3d_32)

    out = _tc4(den3.reshape(NC, NP), acc3.reshape(NC, NP),
               b3.reshape(1, 1))[0]
    return out[0, :N].reshape(N, 1)


# speedup vs baseline: 24.6024x; 1.2814x over previous
import functools

import jax
import jax.numpy as jnp
from jax import lax
from jax.experimental import pallas as pl
from jax.experimental.pallas import tpu as pltpu
from jax.experimental.pallas import tpu_sc as plsc

N = 10000
E = 320000
D = 128

NC = 2
NS = 16
L = 16
CH = 80

NP = 10240
R = 1024
PT = NP // NS
ZR = 64
D2 = 64

_HIGH = lax.Precision.HIGHEST


def _dot(a, b):
    return jnp.dot(a, b, precision=_HIGH, preferred_element_type=jnp.float32)


def _tc1_body(xi, xj, w1, asr, adr, ht, hb, aux):
    h = _dot(xi[...], w1[:D, :]) + _dot(xj[...], w1[D:, :])
    ht[...] = h[:, :D]
    hb[...] = h[:, D:]
    a = jnp.sum(h * asr[...], axis=1)
    b = jnp.sum(h * adr[...], axis=1)
    aux[...] = jnp.concatenate(
        [a[None, :], b[None, :], jnp.zeros((6, R), jnp.float32)], axis=0)


def _tc1(xi, xj, w1, asr, adr):
    return pl.pallas_call(
        _tc1_body,
        grid=(NP // R,),
        in_specs=[
            pl.BlockSpec((R, D), lambda i: (i, 0)),
            pl.BlockSpec((R, D), lambda i: (i, 0)),
            pl.BlockSpec((2 * D, 2 * D), lambda i: (0, 0)),
            pl.BlockSpec((1, 2 * D), lambda i: (0, 0)),
            pl.BlockSpec((1, 2 * D), lambda i: (0, 0)),
        ],
        out_specs=[
            pl.BlockSpec((R, D), lambda i: (i, 0)),
            pl.BlockSpec((R, D), lambda i: (i, 0)),
            pl.BlockSpec((8, R), lambda i: (0, i)),
        ],
        out_shape=[
            jax.ShapeDtypeStruct((NP, D), jnp.float32),
            jax.ShapeDtypeStruct((NP, D), jnp.float32),
            jax.ShapeDtypeStruct((8, NP), jnp.float32),
        ],
    )(xi, xj, w1, asr, adr)


def _tc2_body(accA, accB, den, b1, w2, asr, adr, h2, aux):
    d = den[0, :][:, None] + 1e-16
    x2a = jax.nn.relu(accA[...] / d + b1[0, :D])
    x2b = jax.nn.relu(accB[...] / d + b1[0, D:])
    h = _dot(x2a, w2[:D, :]) + _dot(x2b, w2[D:, :])
    h2[...] = h
    a = jnp.sum(h * asr[...], axis=1)
    b = jnp.sum(h * adr[...], axis=1)
    aux[...] = jnp.concatenate(
        [a[None, :], b[None, :], jnp.zeros((6, R), jnp.float32)], axis=0)


def _tc2(acc1, den1, b1, w2, asr, adr):
    return pl.pallas_call(
        _tc2_body,
        grid=(NP // R,),
        in_specs=[
            pl.BlockSpec((R, D), lambda i: (i, 0)),
            pl.BlockSpec((R, D), lambda i: (NP // R + i, 0)),
            pl.BlockSpec((1, R), lambda i: (0, i)),
            pl.BlockSpec((1, 2 * D), lambda i: (0, 0)),
            pl.BlockSpec((2 * D, D), lambda i: (0, 0)),
            pl.BlockSpec((1, D), lambda i: (0, 0)),
            pl.BlockSpec((1, D), lambda i: (0, 0)),
        ],
        out_specs=[
            pl.BlockSpec((R, D), lambda i: (i, 0)),
            pl.BlockSpec((8, R), lambda i: (0, i)),
        ],
        out_shape=[
            jax.ShapeDtypeStruct((NP, D), jnp.float32),
            jax.ShapeDtypeStruct((8, NP), jnp.float32),
        ],
    )(acc1, acc1, den1, b1, w2, asr, adr)


def _tc3_body(accA, accB, den, b2, w3, as3, ad3, aux):
    d = (den[0, :] + den[1, :])[:, None] + 1e-16
    x3 = jax.nn.relu((accA[...] + accB[...]) / d + b2[0, :])
    h3 = jnp.sum(x3 * w3[...], axis=1)
    aux[...] = jnp.concatenate(
        [h3[None, :], (h3 * as3[0, 0])[None, :], (h3 * ad3[0, 0])[None, :],
         jnp.zeros((5, R), jnp.float32)], axis=0)


def _tc3(acc2, den2, b2, w3, as3, ad3):
    return pl.pallas_call(
        _tc3_body,
        grid=(NP // R,),
        in_specs=[
            pl.BlockSpec((R, D), lambda i: (i, 0)),
            pl.BlockSpec((R, D), lambda i: (NP // R + i, 0)),
            pl.BlockSpec((2, R), lambda i: (0, i)),
            pl.BlockSpec((1, D), lambda i: (0, 0)),
            pl.BlockSpec((1, D), lambda i: (0, 0)),
            pl.BlockSpec((1, 1), lambda i: (0, 0)),
            pl.BlockSpec((1, 1), lambda i: (0, 0)),
        ],
        out_specs=[pl.BlockSpec((8, R), lambda i: (0, i))],
        out_shape=[jax.ShapeDtypeStruct((8, NP), jnp.float32)],
    )(acc2, acc2, den2, b2, w3, as3, ad3)


def _tc4_body(den, acc, b3, out):
    d = den[0, :] + den[1, :] + 1e-16
    a = acc[0, :] + acc[1, :]
    out[...] = jax.nn.sigmoid(a / d + b3[0, 0])[None, :]


def _tc4(den3, acc3, b3):
    return pl.pallas_call(
        _tc4_body,
        grid=(NP // R,),
        in_specs=[
            pl.BlockSpec((2, R), lambda i: (0, i)),
            pl.BlockSpec((2, R), lambda i: (0, i)),
            pl.BlockSpec((1, 1), lambda i: (0, 0)),
        ],
        out_specs=[pl.BlockSpec((1, R), lambda i: (0, i))],
        out_shape=[jax.ShapeDtypeStruct((1, NP), jnp.float32)],
    )(den3, acc3, b3)


_MESH = plsc.VectorSubcoreMesh(core_axis_name="c", subcore_axis_name="s")
_SC_PARAMS = pltpu.CompilerParams(needs_layout_passes=False)
NB = 5


NH = NP // 2
PT2 = NH // NS


def _edge_body(col_split, htab, aux, src_h, dst_h, acc_out, den_out,
               as_t, ad_t, sidx, rows, gidx, exv, dst_v, dstl,
               zrow_v, zden_v, acc_sp, den_sp, semi, semg, semr, semd):
    c = lax.axis_index("c")
    s = lax.axis_index("s")
    den_cond = (c == 0) if col_split else (c >= 0)
    z16f = jnp.zeros((L,), jnp.float32)
    z16i = jnp.zeros((L,), jnp.int32)

    if col_split:
        ep = E // NS
        base = s * ep
    else:
        ep = E // (NC * NS)
        base = (c * NS + s) * ep
    nch = ep // CH

    pltpu.sync_copy(aux.at[0], as_t)
    pltpu.sync_copy(aux.at[1], ad_t)

    @pl.loop(0, ZR)
    def _zr(i):
        for j in range(D // L):
            zrow_v[i, pl.ds(L * j, L)] = z16f

    @pl.loop(0, PT // L)
    def _zd(i):
        zden_v[pl.ds(pl.multiple_of(L * i, 8), L)] = z16f

    @pl.when(den_cond)
    def _():
        pltpu.sync_copy(zden_v, den_sp.at[pl.ds(s * PT, PT)])

    def issue_idx(b, g):
        eoff = pl.multiple_of(base + g * CH, 8)
        pltpu.async_copy(src_h.at[pl.ds(eoff, CH)], sidx[b], semi[b])
        pltpu.async_copy(dst_h.at[pl.ds(eoff, CH)], dst_v[b], semi[b])

    for p_half in range(2):
        lo = p_half * NH

        for k in range(PT2 // ZR):
            pltpu.sync_copy(zrow_v, acc_sp.at[pl.ds(s * PT2 + k * ZR, ZR)])

        plsc.subcore_barrier()

        def prep(b):
            pltpu.make_async_copy(src_h.at[pl.ds(0, CH)], sidx[b],
                                  semi[b]).wait()
            pltpu.make_async_copy(dst_h.at[pl.ds(0, CH)], dst_v[b],
                                  semi[b]).wait()
            for j in range(CH // L):
                si = sidx[b][pl.ds(L * j, L)]
                di = dst_v[b][pl.ds(L * j, L)]
                av = plsc.load_gather(as_t, [si])
                bv = plsc.load_gather(ad_t, [di])
                e = av + bv
                e = jnp.where(e >= 0.0, e, 0.2 * e)
                exv[b][pl.ds(L * j, L)] = jnp.exp(e)
                gidx[b][pl.ds(L * j, L)] = (si + c * NP) if col_split else si
                dl = di - lo
                dstl[b][pl.ds(L * j, L)] = jnp.where(
                    (dl >= 0) & (dl < NH), dl, NH)
            pltpu.async_copy(htab.at[gidx[b]], rows[b], semg[b])

        def finish(b):
            pltpu.make_async_copy(htab.at[gidx[b]], rows[b],
                                  semg[b]).wait()

            @pl.loop(0, CH, unroll=4)
            def _row(i):
                m = plsc.load_gather(exv[b], [z16i + i])
                for j2 in range(D // L):
                    rows[b][i, pl.ds(L * j2, L)] = (
                        rows[b][i, pl.ds(L * j2, L)] * m)

            pltpu.async_copy(rows[b], acc_sp.at[dstl[b]], semr[b],
                             add=True)
            if p_half == 0:
                @pl.when(den_cond)
                def _():
                    pltpu.async_copy(exv[b], den_sp.at[dst_v[b]], semd[b],
                                     add=True)

        def drain(b):
            pltpu.make_async_copy(rows[b], acc_sp.at[dstl[b]],
                                  semr[b]).wait()
            if p_half == 0:
                @pl.when(den_cond)
                def _():
                    pltpu.make_async_copy(exv[b], den_sp.at[dst_v[b]],
                                          semd[b]).wait()

        for b in range(NB):
            issue_idx(b, b)
        for b in range(NB):
            prep(b)

        @pl.loop(0, nch // NB - 1)
        def _grp(p):
            g0 = p * NB
            for b in range(NB):
                finish(b)
                issue_idx(b, g0 + NB + b)
            for b in range(NB):
                drain(b)
                prep(b)

        for b in range(NB):
            finish(b)
        for b in range(NB):
            drain(b)

        plsc.subcore_barrier()

        for k in range(PT2 // ZR):
            pltpu.sync_copy(
                acc_sp.at[pl.ds(s * PT2 + k * ZR, ZR)],
                acc_out.at[pl.ds(c * NP + lo + s * PT2 + k * ZR, ZR)])

        if p_half == 0:
            @pl.when(den_cond)
            def _():
                if col_split:
                    pltpu.sync_copy(den_sp.at[pl.ds(s * PT, PT)],
                                    den_out.at[pl.ds(s * PT, PT)])
                else:
                    pltpu.sync_copy(den_sp.at[pl.ds(s * PT, PT)],
                                    den_out.at[pl.ds(c * NP + s * PT, PT)])


def _make_edge(col_split):
    den_len = NP if col_split else NC * NP
    return pl.kernel(
        functools.partial(_edge_body, col_split),
        out_type=[
            jax.ShapeDtypeStruct((NC * NP, D), jnp.float32),
            jax.ShapeDtypeStruct((den_len,), jnp.float32),
        ],
        mesh=_MESH,
        compiler_params=_SC_PARAMS,
        scratch_types=[
            pltpu.VMEM((NP,), jnp.float32),
            pltpu.VMEM((NP,), jnp.float32),
            tuple(pltpu.VMEM((CH,), jnp.int32) for _ in range(NB)),
            tuple(pltpu.VMEM((CH, D), jnp.float32) for _ in range(NB)),
            tuple(pltpu.VMEM((CH,), jnp.int32) for _ in range(NB)),
            tuple(pltpu.VMEM((CH,), jnp.float32) for _ in range(NB)),
            tuple(pltpu.VMEM((CH,), jnp.int32) for _ in range(NB)),
            tuple(pltpu.VMEM((CH,), jnp.int32) for _ in range(NB)),
            pltpu.VMEM((ZR, D), jnp.float32),
            pltpu.VMEM((PT,), jnp.float32),
            pltpu.VMEM_SHARED((NH + 8, D), jnp.float32),
            pltpu.VMEM_SHARED((NP,), jnp.float32),
            tuple(pltpu.SemaphoreType.DMA for _ in range(NB)),
            tuple(pltpu.SemaphoreType.DMA for _ in range(NB)),
            tuple(pltpu.SemaphoreType.DMA for _ in range(NB)),
            tuple(pltpu.SemaphoreType.DMA for _ in range(NB)),
        ],
    )


_sc_l1 = _make_edge(col_split=True)
_sc_l2 = _make_edge(col_split=False)


_L3_EP = E // (NC * NS)
_L3_NCH = _L3_EP // CH


def _sc_l3_body(aux, src_h, dst3d_h, den_out, acc_out,
                as_t, ad_t, h_t, srcb, dstb, exv, pv, dst_v, zden_v,
                den_sp, acc_sp, semd, sema):
    c = lax.axis_index("c")
    s = lax.axis_index("s")
    z16f = jnp.zeros((L,), jnp.float32)

    base = (c * NS + s) * _L3_EP

    pltpu.sync_copy(aux.at[0], h_t)
    pltpu.sync_copy(aux.at[1], as_t)
    pltpu.sync_copy(aux.at[2], ad_t)
    pltpu.sync_copy(src_h.at[pl.ds(pl.multiple_of(base, 8), _L3_EP)], srcb)
    pltpu.sync_copy(dst3d_h.at[c * NS + s], dstb)

    @pl.loop(0, PT // L)
    def _zd(i):
        zden_v[pl.ds(pl.multiple_of(L * i, 8), L)] = z16f

    pltpu.sync_copy(zden_v, den_sp.at[pl.ds(s * PT, PT)])
    pltpu.sync_copy(zden_v, acc_sp.at[pl.ds(s * PT, PT)])
    plsc.subcore_barrier()

    def prep3(b, g):
        for j in range(CH // L):
            si = srcb[pl.ds(g * CH + L * j, L)]
            di = dstb[g, pl.ds(L * j, L)]
            av = plsc.load_gather(as_t, [si])
            bv = plsc.load_gather(ad_t, [di])
            hv = plsc.load_gather(h_t, [si])
            e = av + bv
            e = jnp.where(e >= 0.0, e, 0.2 * e)
            ex = jnp.exp(e)
            exv[b][pl.ds(L * j, L)] = ex
            pv[b][pl.ds(L * j, L)] = ex * hv
            dst_v[b][pl.ds(L * j, L)] = di
        pltpu.async_copy(exv[b], den_sp.at[dst_v[b]], semd[b], add=True)
        pltpu.async_copy(pv[b], acc_sp.at[dst_v[b]], sema[b], add=True)

    def drain3(b):
        pltpu.make_async_copy(exv[b], den_sp.at[dst_v[b]], semd[b]).wait()
        pltpu.make_async_copy(pv[b], acc_sp.at[dst_v[b]], sema[b]).wait()

    for b in range(NB):
        prep3(b, b)

    @pl.loop(0, _L3_NCH // NB - 1)
    def _grp(p):
        g0 = p * NB
        for b in range(NB):
            drain3(b)
            prep3(b, g0 + NB + b)

    for b in range(NB):
        drain3(b)

    plsc.subcore_barrier()
    pltpu.sync_copy(den_sp.at[pl.ds(s * PT, PT)],
                    den_out.at[pl.ds(c * NP + s * PT, PT)])
    pltpu.sync_copy(acc_sp.at[pl.ds(s * PT, PT)],
                    acc_out.at[pl.ds(c * NP + s * PT, PT)])


_sc_l3 = pl.kernel(
    _sc_l3_body,
    out_type=[
        jax.ShapeDtypeStruct((NC * NP,), jnp.float32),
        jax.ShapeDtypeStruct((NC * NP,), jnp.float32),
    ],
    mesh=_MESH,
    compiler_params=_SC_PARAMS,
    scratch_types=[
        pltpu.VMEM((NP,), jnp.float32),
        pltpu.VMEM((NP,), jnp.float32),
        pltpu.VMEM((NP,), jnp.float32),
        pltpu.VMEM((_L3_EP,), jnp.int32),
        pltpu.VMEM((_L3_NCH, CH), jnp.int32),
        tuple(pltpu.VMEM((CH,), jnp.float32) for _ in range(NB)),
        tuple(pltpu.VMEM((CH,), jnp.float32) for _ in range(NB)),
        tuple(pltpu.VMEM((CH,), jnp.int32) for _ in range(NB)),
        pltpu.VMEM((PT,), jnp.float32),
        pltpu.VMEM_SHARED((NP,), jnp.float32),
        pltpu.VMEM_SHARED((NP,), jnp.float32),
        tuple(pltpu.SemaphoreType.DMA for _ in range(NB)),
        tuple(pltpu.SemaphoreType.DMA for _ in range(NB)),
    ],
)


def kernel(x_i, x_j, edge_index, W1, a_src1, a_dst1, b1,
           W2, a_src2, a_dst2, b2, W3, a_src3, a_dst3, b3):
    src = edge_index[0]
    dst = edge_index[1]
    dst3d_32 = dst.reshape(NC * NS, -1, CH)
    pad = ((0, NP - N), (0, 0))
    xi = jnp.pad(x_i, pad)
    xj = jnp.pad(x_j, pad)

    ht, hb, aux1 = _tc1(xi, xj, W1,
                        a_src1.reshape(1, -1), a_dst1.reshape(1, -1))
    htab1 = jnp.concatenate([ht, hb], axis=0)
    acc1, den1 = _sc_l1(htab1, aux1, src, dst)

    h2, aux2 = _tc2(acc1, den1.reshape(1, NP), b1.reshape(1, -1),
                    W2, a_src2.reshape(1, -1), a_dst2.reshape(1, -1))
    acc2, den2 = _sc_l2(h2, aux2, src, dst)

    aux3 = _tc3(acc2, den2.reshape(NC, NP), b2.reshape(1, -1),
                W3.reshape(1, D), a_src3.reshape(1, 1), a_dst3.reshape(1, 1))[0]
    den3, acc3 = _sc_l3(aux3, src, dst3d_32)

    out = _tc4(den3.reshape(NC, NP), acc3.reshape(NC, NP),
               b3.reshape(1, 1))[0]
    return out[0, :N].reshape(N, 1)


# speedup vs baseline: 30.5325x; 1.2410x over previous
import functools

import jax
import jax.numpy as jnp
from jax import lax
from jax.experimental import pallas as pl
from jax.experimental.pallas import tpu as pltpu
from jax.experimental.pallas import tpu_sc as plsc

N = 10000
E = 320000
D = 128

NC = 2
NS = 16
L = 16
CH = 80

NP = 10240
R = 1024
PT = NP // NS
ZR = 64
D2 = 64

_HIGH = lax.Precision.HIGHEST


def _dot(a, b):
    return jnp.dot(a, b, precision=_HIGH, preferred_element_type=jnp.float32)


def _tc1_body(xi, xj, w1, asr, adr, ht, hb, aux):
    h = _dot(xi[...], w1[:D, :]) + _dot(xj[...], w1[D:, :])
    ht[...] = h[:, :D]
    hb[...] = h[:, D:]
    a = jnp.sum(h * asr[...], axis=1)
    b = jnp.sum(h * adr[...], axis=1)
    aux[...] = jnp.concatenate(
        [a[None, :], b[None, :], jnp.zeros((6, R), jnp.float32)], axis=0)


def _tc1(xi, xj, w1, asr, adr):
    return pl.pallas_call(
        _tc1_body,
        grid=(NP // R,),
        in_specs=[
            pl.BlockSpec((R, D), lambda i: (i, 0)),
            pl.BlockSpec((R, D), lambda i: (i, 0)),
            pl.BlockSpec((2 * D, 2 * D), lambda i: (0, 0)),
            pl.BlockSpec((1, 2 * D), lambda i: (0, 0)),
            pl.BlockSpec((1, 2 * D), lambda i: (0, 0)),
        ],
        out_specs=[
            pl.BlockSpec((R, D), lambda i: (i, 0)),
            pl.BlockSpec((R, D), lambda i: (i, 0)),
            pl.BlockSpec((8, R), lambda i: (0, i)),
        ],
        out_shape=[
            jax.ShapeDtypeStruct((NP, D), jnp.float32),
            jax.ShapeDtypeStruct((NP, D), jnp.float32),
            jax.ShapeDtypeStruct((8, NP), jnp.float32),
        ],
    )(xi, xj, w1, asr, adr)


def _tc2_body(acc0, acc1, acc2, acc3, den, b1, w2, asr, adr, h2, aux):
    d = den[0, :][:, None] + 1e-16
    accs = (acc0, acc1, acc2, acc3)
    h = jnp.zeros((R, D), jnp.float32)
    for q in range(4):
        xq = jax.nn.relu(accs[q][...] / d + b1[0, q * D2:(q + 1) * D2])
        h = h + _dot(xq, w2[q * D2:(q + 1) * D2, :])
    h2[...] = h
    a = jnp.sum(h * asr[...], axis=1)
    b = jnp.sum(h * adr[...], axis=1)
    aux[...] = jnp.concatenate(
        [a[None, :], b[None, :], jnp.zeros((6, R), jnp.float32)], axis=0)


def _tc2(acc1, den1, b1, w2, asr, adr):
    qspec = [pl.BlockSpec((R, D2), functools.partial(
        lambda q, i: (q * (NP // R) + i, 0), q)) for q in range(4)]
    return pl.pallas_call(
        _tc2_body,
        grid=(NP // R,),
        in_specs=qspec + [
            pl.BlockSpec((1, R), lambda i: (0, i)),
            pl.BlockSpec((1, 2 * D), lambda i: (0, 0)),
            pl.BlockSpec((2 * D, D), lambda i: (0, 0)),
            pl.BlockSpec((1, D), lambda i: (0, 0)),
            pl.BlockSpec((1, D), lambda i: (0, 0)),
        ],
        out_specs=[
            pl.BlockSpec((R, D), lambda i: (i, 0)),
            pl.BlockSpec((8, R), lambda i: (0, i)),
        ],
        out_shape=[
            jax.ShapeDtypeStruct((NP, D), jnp.float32),
            jax.ShapeDtypeStruct((8, NP), jnp.float32),
        ],
    )(acc1, acc1, acc1, acc1, den1, b1, w2, asr, adr)


def _tc3_body(acc0, acc1, acc2, acc3, den, b2, w3, as3, ad3, aux):
    d = (den[0, :] + den[1, :])[:, None] + 1e-16
    accs = (acc0, acc1, acc2, acc3)
    h3 = jnp.zeros((R,), jnp.float32)
    for p in range(2):
        xp = jax.nn.relu((accs[2 * p][...] + accs[2 * p + 1][...]) / d
                         + b2[0, p * D2:(p + 1) * D2])
        h3 = h3 + jnp.sum(xp * w3[0, p * D2:(p + 1) * D2], axis=1)
    aux[...] = jnp.concatenate(
        [h3[None, :], (h3 * as3[0, 0])[None, :], (h3 * ad3[0, 0])[None, :],
         jnp.zeros((5, R), jnp.float32)], axis=0)


def _tc3(acc2, den2, b2, w3, as3, ad3):
    qspec = [pl.BlockSpec((R, D2), functools.partial(
        lambda q, i: (q * (NP // R) + i, 0), q)) for q in range(4)]
    return pl.pallas_call(
        _tc3_body,
        grid=(NP // R,),
        in_specs=qspec + [
            pl.BlockSpec((2, R), lambda i: (0, i)),
            pl.BlockSpec((1, D), lambda i: (0, 0)),
            pl.BlockSpec((1, D), lambda i: (0, 0)),
            pl.BlockSpec((1, 1), lambda i: (0, 0)),
            pl.BlockSpec((1, 1), lambda i: (0, 0)),
        ],
        out_specs=[pl.BlockSpec((8, R), lambda i: (0, i))],
        out_shape=[jax.ShapeDtypeStruct((8, NP), jnp.float32)],
    )(acc2, acc2, acc2, acc2, den2, b2, w3, as3, ad3)


def _tc4_body(den, acc, b3, out):
    d = den[0, :] + den[1, :] + 1e-16
    a = acc[0, :] + acc[1, :]
    out[...] = jax.nn.sigmoid(a / d + b3[0, 0])[None, :]


def _tc4(den3, acc3, b3):
    return pl.pallas_call(
        _tc4_body,
        grid=(NP // R,),
        in_specs=[
            pl.BlockSpec((2, R), lambda i: (0, i)),
            pl.BlockSpec((2, R), lambda i: (0, i)),
            pl.BlockSpec((1, 1), lambda i: (0, 0)),
        ],
        out_specs=[pl.BlockSpec((1, R), lambda i: (0, i))],
        out_shape=[jax.ShapeDtypeStruct((1, NP), jnp.float32)],
    )(den3, acc3, b3)


_MESH = plsc.VectorSubcoreMesh(core_axis_name="c", subcore_axis_name="s")
_SC_PARAMS = pltpu.CompilerParams(needs_layout_passes=False,
                                  use_tc_tiling_on_sc=False)
NB = 5


def _edge_body(col_split, htab, aux, src_h, dst_h, acc_out, den_out,
               as_t, ad_t, sidx, rows, gidx, exv, dst_v,
               zrow_v, zden_v, acc_sp, den_sp, semi, semg, semr, semd):
    c = lax.axis_index("c")
    s = lax.axis_index("s")
    den_cond = (c == 0) if col_split else (c >= 0)
    z16f = jnp.zeros((L,), jnp.float32)
    z16i = jnp.zeros((L,), jnp.int32)

    if col_split:
        ep = E // NS
        base = s * ep
    else:
        ep = E // (NC * NS)
        base = (c * NS + s) * ep
    nch = ep // CH

    pltpu.sync_copy(aux.at[0], as_t)
    pltpu.sync_copy(aux.at[1], ad_t)

    @pl.loop(0, ZR)
    def _zr(i):
        for j in range(D2 // L):
            zrow_v[i, pl.ds(L * j, L)] = z16f

    @pl.loop(0, PT // L)
    def _zd(i):
        zden_v[pl.ds(pl.multiple_of(L * i, 8), L)] = z16f

    @pl.when(den_cond)
    def _():
        pltpu.sync_copy(zden_v, den_sp.at[pl.ds(s * PT, PT)])

    def issue_idx(b, g):
        eoff = pl.multiple_of(base + g * CH, 8)
        pltpu.async_copy(src_h.at[pl.ds(eoff, CH)], sidx[b], semi[b])
        pltpu.async_copy(dst_h.at[pl.ds(eoff, CH)], dst_v[b], semi[b])

    for p_col in range(2):
        qrow = (c * 2 + p_col) * NP if col_split else (p_col * NC + c) * NP
        toff = (c * 2 + p_col) * NP if col_split else p_col * NP

        for k in range(PT // ZR):
            pltpu.sync_copy(zrow_v, acc_sp.at[pl.ds(s * PT + k * ZR, ZR)])

        plsc.subcore_barrier()

        def prep(b):
            pltpu.make_async_copy(src_h.at[pl.ds(0, CH)], sidx[b],
                                  semi[b]).wait()
            pltpu.make_async_copy(dst_h.at[pl.ds(0, CH)], dst_v[b],
                                  semi[b]).wait()
            for j in range(CH // L):
                si = sidx[b][pl.ds(L * j, L)]
                di = dst_v[b][pl.ds(L * j, L)]
                av = plsc.load_gather(as_t, [si])
                bv = plsc.load_gather(ad_t, [di])
                e = av + bv
                e = jnp.where(e >= 0.0, e, 0.2 * e)
                exv[b][pl.ds(L * j, L)] = jnp.exp(e)
                gidx[b][pl.ds(L * j, L)] = si + toff
            pltpu.async_copy(htab.at[gidx[b]], rows[b], semg[b])

        def finish(b):
            pltpu.make_async_copy(htab.at[gidx[b]], rows[b],
                                  semg[b]).wait()

            @pl.loop(0, CH, unroll=4)
            def _row(i):
                m = plsc.load_gather(exv[b], [z16i + i])
                for j2 in range(D2 // L):
                    rows[b][i, pl.ds(L * j2, L)] = (
                        rows[b][i, pl.ds(L * j2, L)] * m)

            pltpu.async_copy(rows[b], acc_sp.at[dst_v[b]], semr[b],
                             add=True)
            if p_col == 0:
                @pl.when(den_cond)
                def _():
                    pltpu.async_copy(exv[b], den_sp.at[dst_v[b]], semd[b],
                                     add=True)

        def drain(b):
            pltpu.make_async_copy(rows[b], acc_sp.at[dst_v[b]],
                                  semr[b]).wait()
            if p_col == 0:
                @pl.when(den_cond)
                def _():
                    pltpu.make_async_copy(exv[b], den_sp.at[dst_v[b]],
                                          semd[b]).wait()

        for b in range(NB):
            issue_idx(b, b)
        for b in range(NB):
            prep(b)

        @pl.loop(0, nch // NB - 1)
        def _grp(p):
            g0 = p * NB
            for b in range(NB):
                finish(b)
                issue_idx(b, g0 + NB + b)
            for b in range(NB):
                drain(b)
                prep(b)

        for b in range(NB):
            finish(b)
        for b in range(NB):
            drain(b)

        plsc.subcore_barrier()

        for k in range(PT // ZR):
            pltpu.sync_copy(
                acc_sp.at[pl.ds(s * PT + k * ZR, ZR)],
                acc_out.at[pl.ds(qrow + s * PT + k * ZR, ZR)])

        if p_col == 0:
            @pl.when(den_cond)
            def _():
                if col_split:
                    pltpu.sync_copy(den_sp.at[pl.ds(s * PT, PT)],
                                    den_out.at[pl.ds(s * PT, PT)])
                else:
                    pltpu.sync_copy(den_sp.at[pl.ds(s * PT, PT)],
                                    den_out.at[pl.ds(c * NP + s * PT, PT)])


def _make_edge(col_split):
    den_len = NP if col_split else NC * NP
    return pl.kernel(
        functools.partial(_edge_body, col_split),
        out_type=[
            jax.ShapeDtypeStruct((4 * NP, D2), jnp.float32),
            jax.ShapeDtypeStruct((den_len,), jnp.float32),
        ],
        mesh=_MESH,
        compiler_params=_SC_PARAMS,
        scratch_types=[
            pltpu.VMEM((NP,), jnp.float32),
            pltpu.VMEM((NP,), jnp.float32),
            tuple(pltpu.VMEM((CH,), jnp.int32) for _ in range(NB)),
            tuple(pltpu.VMEM((CH, D2), jnp.float32) for _ in range(NB)),
            tuple(pltpu.VMEM((CH,), jnp.int32) for _ in range(NB)),
            tuple(pltpu.VMEM((CH,), jnp.float32) for _ in range(NB)),
            tuple(pltpu.VMEM((CH,), jnp.int32) for _ in range(NB)),
            pltpu.VMEM((ZR, D2), jnp.float32),
            pltpu.VMEM((PT,), jnp.float32),
            pltpu.VMEM_SHARED((NP, D2), jnp.float32),
            pltpu.VMEM_SHARED((NP,), jnp.float32),
            tuple(pltpu.SemaphoreType.DMA for _ in range(NB)),
            tuple(pltpu.SemaphoreType.DMA for _ in range(NB)),
            tuple(pltpu.SemaphoreType.DMA for _ in range(NB)),
            tuple(pltpu.SemaphoreType.DMA for _ in range(NB)),
        ],
    )


_sc_l1 = _make_edge(col_split=True)
_sc_l2 = _make_edge(col_split=False)


_L3_EP = E // (NC * NS)
_L3_NCH = _L3_EP // CH


def _sc_l3_body(aux, src_h, dst3d_h, den_out, acc_out,
                as_t, ad_t, h_t, srcb, dstb, exv, pv, dst_v, zden_v,
                den_sp, acc_sp, semd, sema):
    c = lax.axis_index("c")
    s = lax.axis_index("s")
    z16f = jnp.zeros((L,), jnp.float32)

    base = (c * NS + s) * _L3_EP

    pltpu.sync_copy(aux.at[0], h_t)
    pltpu.sync_copy(aux.at[1], as_t)
    pltpu.sync_copy(aux.at[2], ad_t)
    pltpu.sync_copy(src_h.at[pl.ds(pl.multiple_of(base, 8), _L3_EP)], srcb)
    pltpu.sync_copy(dst3d_h.at[c * NS + s], dstb)

    @pl.loop(0, PT // L)
    def _zd(i):
        zden_v[pl.ds(pl.multiple_of(L * i, 8), L)] = z16f

    pltpu.sync_copy(zden_v, den_sp.at[pl.ds(s * PT, PT)])
    pltpu.sync_copy(zden_v, acc_sp.at[pl.ds(s * PT, PT)])
    plsc.subcore_barrier()

    def prep3(b, g):
        for j in range(CH // L):
            si = srcb[pl.ds(g * CH + L * j, L)]
            di = dstb[g, pl.ds(L * j, L)]
            av = plsc.load_gather(as_t, [si])
            bv = plsc.load_gather(ad_t, [di])
            hv = plsc.load_gather(h_t, [si])
            e = av + bv
            e = jnp.where(e >= 0.0, e, 0.2 * e)
            ex = jnp.exp(e)
            exv[b][pl.ds(L * j, L)] = ex
            pv[b][pl.ds(L * j, L)] = ex * hv
            dst_v[b][pl.ds(L * j, L)] = di
        pltpu.async_copy(exv[b], den_sp.at[dst_v[b]], semd[b], add=True)
        pltpu.async_copy(pv[b], acc_sp.at[dst_v[b]], sema[b], add=True)

    def drain3(b):
        pltpu.make_async_copy(exv[b], den_sp.at[dst_v[b]], semd[b]).wait()
        pltpu.make_async_copy(pv[b], acc_sp.at[dst_v[b]], sema[b]).wait()

    for b in range(NB):
        prep3(b, b)

    @pl.loop(0, _L3_NCH // NB - 1)
    def _grp(p):
        g0 = p * NB
        for b in range(NB):
            drain3(b)
            prep3(b, g0 + NB + b)

    for b in range(NB):
        drain3(b)

    plsc.subcore_barrier()
    pltpu.sync_copy(den_sp.at[pl.ds(s * PT, PT)],
                    den_out.at[pl.ds(c * NP + s * PT, PT)])
    pltpu.sync_copy(acc_sp.at[pl.ds(s * PT, PT)],
                    acc_out.at[pl.ds(c * NP + s * PT, PT)])


_sc_l3 = pl.kernel(
    _sc_l3_body,
    out_type=[
        jax.ShapeDtypeStruct((NC * NP,), jnp.float32),
        jax.ShapeDtypeStruct((NC * NP,), jnp.float32),
    ],
    mesh=_MESH,
    compiler_params=_SC_PARAMS,
    scratch_types=[
        pltpu.VMEM((NP,), jnp.float32),
        pltpu.VMEM((NP,), jnp.float32),
        pltpu.VMEM((NP,), jnp.float32),
        pltpu.VMEM((_L3_EP,), jnp.int32),
        pltpu.VMEM((_L3_NCH, CH), jnp.int32),
        tuple(pltpu.VMEM((CH,), jnp.float32) for _ in range(NB)),
        tuple(pltpu.VMEM((CH,), jnp.float32) for _ in range(NB)),
        tuple(pltpu.VMEM((CH,), jnp.int32) for _ in range(NB)),
        pltpu.VMEM((PT,), jnp.float32),
        pltpu.VMEM_SHARED((NP,), jnp.float32),
        pltpu.VMEM_SHARED((NP,), jnp.float32),
        tuple(pltpu.SemaphoreType.DMA for _ in range(NB)),
        tuple(pltpu.SemaphoreType.DMA for _ in range(NB)),
    ],
)


def kernel(x_i, x_j, edge_index, W1, a_src1, a_dst1, b1,
           W2, a_src2, a_dst2, b2, W3, a_src3, a_dst3, b3):
    src = edge_index[0]
    dst = edge_index[1]
    dst3d_32 = dst.reshape(NC * NS, -1, CH)
    pad = ((0, NP - N), (0, 0))
    xi = jnp.pad(x_i, pad)
    xj = jnp.pad(x_j, pad)

    ht, hb, aux1 = _tc1(xi, xj, W1,
                        a_src1.reshape(1, -1), a_dst1.reshape(1, -1))
    htab1 = jnp.concatenate(
        [ht[:, :D2], ht[:, D2:], hb[:, :D2], hb[:, D2:]], axis=0)
    acc1, den1 = _sc_l1(htab1, aux1, src, dst)

    h2, aux2 = _tc2(acc1, den1.reshape(1, NP), b1.reshape(1, -1),
                    W2, a_src2.reshape(1, -1), a_dst2.reshape(1, -1))
    htab2 = jnp.concatenate([h2[:, :D2], h2[:, D2:]], axis=0)
    acc2, den2 = _sc_l2(htab2, aux2, src, dst)

    aux3 = _tc3(acc2, den2.reshape(NC, NP), b2.reshape(1, -1),
                W3.reshape(1, D), a_src3.reshape(1, 1), a_dst3.reshape(1, 1))[0]
    den3, acc3 = _sc_l3(aux3, src, dst3d_32)

    out = _tc4(den3.reshape(NC, NP), acc3.reshape(NC, NP),
               b3.reshape(1, 1))[0]
    return out[0, :N].reshape(N, 1)
